# Initial kernel scaffold; baseline (speedup 1.0000x reference)
import jax
import jax.numpy as jnp
from jax.experimental import pallas as pl


def kernel(input_ids, word_emb, pos_emb, type_emb, ln_gamma, ln_beta):
    raise NotImplementedError("write your pallas kernel here")


# speedup vs baseline: 1.7485x; 1.7485x over previous
import functools

import jax
import jax.numpy as jnp
from jax import lax
from jax.experimental import pallas as pl
from jax.experimental.pallas import tpu as pltpu
from jax.experimental.pallas import tpu_sc as plsc

VOCAB = 30522
HIDDEN = 768
MAX_POS = 2048
BATCH = 4
SEQ = 2048
EPS = 1e-12

NTOK = BATCH * SEQ

_NC, _NS = 2, 16
_NW = _NC * _NS
_TOK_PER_W = NTOK // _NW
_CHUNK = 128
_CHUNKS_PER_W = _TOK_PER_W // _CHUNK


def _sc_gather_body(ids_hbm, table_hbm, out_hbm, idx_v, rows_v, sem):
    wid = lax.axis_index("s") * _NC + lax.axis_index("c")
    pltpu.sync_copy(ids_hbm.at[pl.ds(wid * _CHUNKS_PER_W, _CHUNKS_PER_W)], idx_v)
    for c in range(_CHUNKS_PER_W):
        pltpu.async_copy(table_hbm.at[idx_v.at[c]], rows_v, sem).wait()
        base = wid * _TOK_PER_W + c * _CHUNK
        pltpu.sync_copy(rows_v, out_hbm.at[pl.ds(base, _CHUNK)])


@functools.cache
def _sc_gather():
    return pl.kernel(
        _sc_gather_body,
        out_type=jax.ShapeDtypeStruct((NTOK, HIDDEN), jnp.float32),
        mesh=plsc.VectorSubcoreMesh(core_axis_name="c", subcore_axis_name="s"),
        scratch_types=[
            pltpu.VMEM((_CHUNKS_PER_W, _CHUNK), jnp.int32),
            pltpu.VMEM((_CHUNK, HIDDEN), jnp.float32),
            pltpu.SemaphoreType.DMA,
        ],
    )


_BLK = 512


def _ln_body(x_ref, pos_ref, type_ref, g_ref, b_ref, o_ref):
    x = x_ref[...] + pos_ref[...] + type_ref[0, :][None, :]
    mean = jnp.mean(x, axis=-1, keepdims=True)
    xc = x - mean
    var = jnp.mean(xc * xc, axis=-1, keepdims=True)
    o_ref[...] = xc * lax.rsqrt(var + EPS) * g_ref[...] + b_ref[...]


@jax.jit
def _ln_call(gathered, pos_emb, type_emb, g2, b2):
    grid = (NTOK // _BLK,)
    return pl.pallas_call(
        _ln_body,
        grid=grid,
        in_specs=[
            pl.BlockSpec((_BLK, HIDDEN), lambda i: (i, 0)),
            pl.BlockSpec((_BLK, HIDDEN), lambda i: (i % (SEQ // _BLK), 0)),
            pl.BlockSpec((2, HIDDEN), lambda i: (0, 0)),
            pl.BlockSpec((1, HIDDEN), lambda i: (0, 0)),
            pl.BlockSpec((1, HIDDEN), lambda i: (0, 0)),
        ],
        out_specs=pl.BlockSpec((_BLK, HIDDEN), lambda i: (i, 0)),
        out_shape=jax.ShapeDtypeStruct((NTOK, HIDDEN), jnp.float32),
    )(gathered, pos_emb, type_emb, g2, b2)


def kernel(input_ids, word_emb, pos_emb, type_emb, ln_gamma, ln_beta):
    ids = input_ids.astype(jnp.int32).reshape(NTOK // _CHUNK, _CHUNK)
    gathered = _sc_gather()(ids, word_emb)
    out = _ln_call(gathered, pos_emb, type_emb,
                   ln_gamma.reshape(1, HIDDEN), ln_beta.reshape(1, HIDDEN))
    return out.reshape(BATCH, SEQ, HIDDEN)
